# baseline (speedup 1.0000x reference)
import jax
import jax.numpy as jnp
from jax.experimental import pallas as pl
from jax.experimental.pallas import tpu as pltpu

B, N, F_IN, F_OUT, K = 4, 2048, 256, 256, 3
CHUNK = 256
NCH = N // CHUNK
NR = (B - 1) * N
XCH = NR // 4

_CD0 = (((0,), (0,)), ((), ()))


def _cheb_kernel(adj_hbm, data_hbm, w_ref, bias_ref, out_hbm,
                 adj_v, x0_v, z1_v, dinv_v, a1_v,
                 xr_buf, outr_buf, out0_buf,
                 x0_sem, adj_sem, xr_sem, outw_sem):
    bias = bias_ref[:]
    w1 = w_ref[1]
    w2x2 = 2.0 * w_ref[2]
    wc = w_ref[0] - w_ref[2]

    x0_copy = pltpu.make_async_copy(data_hbm.at[pl.ds(0, N), :], x0_v, x0_sem)
    x0_copy.start()
    adj_copies = []
    for i in range(NCH):
        c = pltpu.make_async_copy(adj_hbm.at[pl.ds(i * CHUNK, CHUNK), :],
                                  adj_v.at[pl.ds(i * CHUNK, CHUNK), :],
                                  adj_sem.at[i])
        c.start()
        adj_copies.append(c)
    xr_copies = []
    for i in range(4):
        c = pltpu.make_async_copy(data_hbm.at[pl.ds(N + i * XCH, XCH), :],
                                  xr_buf.at[i], xr_sem.at[i])
        c.start()
        xr_copies.append(c)

    x0_copy.wait()
    for i in range(NCH):
        adj_copies[i].wait()
        sl = pl.ds(i * CHUNK, CHUNK)
        blk = adj_v[sl, :]
        rowsum = jnp.sum(blk, axis=1, keepdims=True)
        tile = adj_v[sl, i * CHUNK:(i + 1) * CHUNK]
        r = jax.lax.broadcasted_iota(jnp.int32, (CHUNK, CHUNK), 0)
        c = jax.lax.broadcasted_iota(jnp.int32, (CHUNK, CHUNK), 1)
        diag = jnp.sum(jnp.where(r == c, tile, 0.0), axis=1, keepdims=True)
        deg = rowsum - diag
        dinv = jnp.where(deg > 0, jax.lax.rsqrt(jnp.where(deg > 0, deg, 1.0)),
                         0.0)
        dinv_v[sl, :] = dinv
        a1_v[sl, :] = dinv * dinv * diag
        z1_v[sl, :] = dinv * x0_v[sl, :]

    dinv = dinv_v[:]
    a1 = a1_v[:]
    x0 = x0_v[:]
    m1 = jax.lax.dot_general(adj_v[:], z1_v[:], _CD0,
                             preferred_element_type=jnp.float32)
    t1 = a1 * x0 - dinv * m1
    z2 = dinv * t1
    m2 = jax.lax.dot_general(adj_v[:], z2, _CD0,
                             preferred_element_type=jnp.float32)

    out_copies = []
    for i in range(4):
        xr_copies[i].wait()
        o = jnp.dot(xr_buf[i], wc, preferred_element_type=jnp.float32) + bias
        outr_buf[i] = o
        c = pltpu.make_async_copy(outr_buf.at[i],
                                  out_hbm.at[pl.ds(N + i * XCH, XCH), :],
                                  outw_sem.at[i])
        c.start()
        out_copies.append(c)

    t2 = a1 * t1 - dinv * m2
    out0_buf[:] = (
        jnp.dot(x0, wc, preferred_element_type=jnp.float32)
        + jnp.dot(t1, w1, preferred_element_type=jnp.float32)
        + jnp.dot(t2, w2x2, preferred_element_type=jnp.float32)
        + bias)
    c = pltpu.make_async_copy(out0_buf, out_hbm.at[pl.ds(0, N), :],
                              outw_sem.at[4])
    c.start()
    out_copies.append(c)
    for c in out_copies:
        c.wait()


def kernel(data, adj, W, bias):
    out = pl.pallas_call(
        _cheb_kernel,
        in_specs=[
            pl.BlockSpec(memory_space=pltpu.MemorySpace.HBM),
            pl.BlockSpec(memory_space=pltpu.MemorySpace.HBM),
            pl.BlockSpec(memory_space=pltpu.MemorySpace.VMEM),
            pl.BlockSpec(memory_space=pltpu.MemorySpace.VMEM),
        ],
        out_specs=pl.BlockSpec(memory_space=pltpu.MemorySpace.HBM),
        out_shape=jax.ShapeDtypeStruct((B * N, F_OUT), jnp.float32),
        scratch_shapes=[
            pltpu.VMEM((N, N), jnp.float32),
            pltpu.VMEM((N, F_IN), jnp.float32),
            pltpu.VMEM((N, F_IN), jnp.float32),
            pltpu.VMEM((N, 1), jnp.float32),
            pltpu.VMEM((N, 1), jnp.float32),
            pltpu.VMEM((4, XCH, F_IN), jnp.float32),
            pltpu.VMEM((4, XCH, F_OUT), jnp.float32),
            pltpu.VMEM((N, F_OUT), jnp.float32),
            pltpu.SemaphoreType.DMA,
            pltpu.SemaphoreType.DMA((NCH,)),
            pltpu.SemaphoreType.DMA((4,)),
            pltpu.SemaphoreType.DMA((5,)),
        ],
    )(adj, data.reshape(B * N, F_IN), W, bias.reshape(1, F_OUT))
    return out.reshape(B, N, F_OUT)


# speedup vs baseline: 1.0296x; 1.0296x over previous
import jax
import jax.numpy as jnp
from jax.experimental import pallas as pl
from jax.experimental.pallas import tpu as pltpu

B, N, F_IN, F_OUT, K = 4, 2048, 256, 256, 3
CHUNK = 256
NCH = N // CHUNK
NR = (B - 1) * N
XCH = NR // 4

_CD0 = (((0,), (0,)), ((), ()))


def _cheb_kernel(adj_hbm, data_hbm, w_ref, bias_ref, out_hbm,
                 adj_v, x0_v, z1_v, dinv_v, a1_v,
                 xr_buf, outr_buf, out0_buf,
                 x0_sem, adj_sem, xr_sem, outw_sem):
    bias = bias_ref[:]
    w1 = w_ref[1]
    w2x2 = 2.0 * w_ref[2]
    wc = w_ref[0] - w_ref[2]

    x0_copy = pltpu.make_async_copy(data_hbm.at[pl.ds(0, N), :], x0_v, x0_sem)
    x0_copy.start()
    adj_copies = []
    for i in range(NCH):
        c = pltpu.make_async_copy(adj_hbm.at[pl.ds(i * CHUNK, CHUNK), :],
                                  adj_v.at[pl.ds(i * CHUNK, CHUNK), :],
                                  adj_sem.at[i])
        c.start()
        adj_copies.append(c)
    xr_copies = []
    for i in range(4):
        c = pltpu.make_async_copy(data_hbm.at[pl.ds(N + i * XCH, XCH), :],
                                  xr_buf.at[i], xr_sem.at[i])
        c.start()
        xr_copies.append(c)

    out_copies = []
    for i in range(4):
        xr_copies[i].wait()
        o = jnp.dot(xr_buf[i], wc, preferred_element_type=jnp.float32) + bias
        outr_buf[i] = o
        c = pltpu.make_async_copy(outr_buf.at[i],
                                  out_hbm.at[pl.ds(N + i * XCH, XCH), :],
                                  outw_sem.at[i])
        c.start()
        out_copies.append(c)

    x0_copy.wait()
    for i in range(NCH):
        adj_copies[i].wait()
        sl = pl.ds(i * CHUNK, CHUNK)
        blk = adj_v[sl, :]
        rowsum = jnp.sum(blk, axis=1, keepdims=True)
        tile = adj_v[sl, i * CHUNK:(i + 1) * CHUNK]
        r = jax.lax.broadcasted_iota(jnp.int32, (CHUNK, CHUNK), 0)
        c = jax.lax.broadcasted_iota(jnp.int32, (CHUNK, CHUNK), 1)
        diag = jnp.sum(jnp.where(r == c, tile, 0.0), axis=1, keepdims=True)
        deg = rowsum - diag
        dinv = jnp.where(deg > 0, jax.lax.rsqrt(jnp.where(deg > 0, deg, 1.0)),
                         0.0)
        dinv_v[sl, :] = dinv
        a1_v[sl, :] = dinv * dinv * diag
        z1_v[sl, :] = dinv * x0_v[sl, :]

    dinv = dinv_v[:]
    a1 = a1_v[:]
    x0 = x0_v[:]
    m1 = jax.lax.dot_general(adj_v[:], z1_v[:], _CD0,
                             preferred_element_type=jnp.float32)
    t1 = a1 * x0 - dinv * m1
    z2 = dinv * t1
    m2 = jax.lax.dot_general(adj_v[:], z2, _CD0,
                             preferred_element_type=jnp.float32)
    t2 = a1 * t1 - dinv * m2
    out0_buf[:] = (
        jnp.dot(x0, wc, preferred_element_type=jnp.float32)
        + jnp.dot(t1, w1, preferred_element_type=jnp.float32)
        + jnp.dot(t2, w2x2, preferred_element_type=jnp.float32)
        + bias)
    c = pltpu.make_async_copy(out0_buf, out_hbm.at[pl.ds(0, N), :],
                              outw_sem.at[4])
    c.start()
    out_copies.append(c)
    for c in out_copies:
        c.wait()


def kernel(data, adj, W, bias):
    out = pl.pallas_call(
        _cheb_kernel,
        in_specs=[
            pl.BlockSpec(memory_space=pltpu.MemorySpace.HBM),
            pl.BlockSpec(memory_space=pltpu.MemorySpace.HBM),
            pl.BlockSpec(memory_space=pltpu.MemorySpace.VMEM),
            pl.BlockSpec(memory_space=pltpu.MemorySpace.VMEM),
        ],
        out_specs=pl.BlockSpec(memory_space=pltpu.MemorySpace.HBM),
        out_shape=jax.ShapeDtypeStruct((B * N, F_OUT), jnp.float32),
        scratch_shapes=[
            pltpu.VMEM((N, N), jnp.float32),
            pltpu.VMEM((N, F_IN), jnp.float32),
            pltpu.VMEM((N, F_IN), jnp.float32),
            pltpu.VMEM((N, 1), jnp.float32),
            pltpu.VMEM((N, 1), jnp.float32),
            pltpu.VMEM((4, XCH, F_IN), jnp.float32),
            pltpu.VMEM((4, XCH, F_OUT), jnp.float32),
            pltpu.VMEM((N, F_OUT), jnp.float32),
            pltpu.SemaphoreType.DMA,
            pltpu.SemaphoreType.DMA((NCH,)),
            pltpu.SemaphoreType.DMA((4,)),
            pltpu.SemaphoreType.DMA((5,)),
        ],
    )(adj, data.reshape(B * N, F_IN), W, bias.reshape(1, F_OUT))
    return out.reshape(B, N, F_OUT)


# speedup vs baseline: 1.0536x; 1.0233x over previous
import jax
import jax.numpy as jnp
from jax.experimental import pallas as pl

B, N, F_IN, F_OUT, K = 4, 2048, 256, 256, 3
TILE = 256

_CD0 = (((0,), (0,)), ((), ()))


def _cheb_kernel(data_ref, adj_ref, w_ref, bias_ref, out_ref):
    adj = adj_ref[:]
    x = data_ref[:].reshape(B * N, F_IN)
    x0 = x[:N]
    bias = bias_ref[:]

    rowsum = jnp.sum(adj, axis=1, keepdims=True)
    r = jax.lax.broadcasted_iota(jnp.int32, (TILE, TILE), 0)
    c = jax.lax.broadcasted_iota(jnp.int32, (TILE, TILE), 1)
    eye = r == c
    diag_parts = []
    for i in range(N // TILE):
        tile = adj[i * TILE:(i + 1) * TILE, i * TILE:(i + 1) * TILE]
        diag_parts.append(
            jnp.sum(jnp.where(eye, tile, 0.0), axis=1, keepdims=True))
    diag = jnp.concatenate(diag_parts, axis=0)

    deg = rowsum - diag
    dinv = jnp.where(deg > 0, jax.lax.rsqrt(jnp.where(deg > 0, deg, 1.0)), 0.0)
    a1 = dinv * dinv * diag

    z1 = dinv * x0
    m1 = jax.lax.dot_general(adj, z1, _CD0,
                             preferred_element_type=jnp.float32)
    t1 = a1 * x0 - dinv * m1
    z2 = dinv * t1
    m2 = jax.lax.dot_general(adj, z2, _CD0,
                             preferred_element_type=jnp.float32)
    t2 = a1 * t1 - dinv * m2

    wc = w_ref[0] - w_ref[2]
    out_ref[:N, :] = (
        jnp.dot(x0, wc, preferred_element_type=jnp.float32)
        + jnp.dot(t1, w_ref[1], preferred_element_type=jnp.float32)
        + jnp.dot(t2, 2.0 * w_ref[2], preferred_element_type=jnp.float32)
        + bias)
    out_ref[N:, :] = jnp.dot(x[N:], wc,
                             preferred_element_type=jnp.float32) + bias


def kernel(data, adj, W, bias):
    out = pl.pallas_call(
        _cheb_kernel,
        out_shape=jax.ShapeDtypeStruct((B * N, F_OUT), jnp.float32),
    )(data, adj, W, bias.reshape(1, F_OUT))
    return out.reshape(B, N, F_OUT)
